# baseline (speedup 1.0000x reference)
import functools

import jax
import jax.numpy as jnp
from jax import lax
from jax.experimental import pallas as pl
from jax.experimental.pallas import tpu as pltpu
from jax.experimental.pallas import tpu_sc as plsc

N_ATTRIBUTES = 26
N_VALUES = 100000
LOG = 17
BATCH = 16384
D_OUT = N_ATTRIBUTES * LOG

NUM_CORES = 2
NUM_SUBCORES = 16

PACK_BV = 8192
SPLIT_BLOCKS = 6
V_SPLIT = SPLIT_BLOCKS * PACK_BV
V_LO = V_SPLIT
V_HI = N_VALUES - V_SPLIT


def _pack(tab3, block_lo, n_values):
    nblk = -(-n_values // PACK_BV)

    def body(t_ref, p_ref):
        acc = t_ref[0]
        for c in range(1, LOG):
            acc += t_ref[c] * jnp.float32(1 << c)
        p_ref[...] = acc.astype(jnp.int32)

    return pl.pallas_call(
        body,
        grid=(nblk,),
        in_specs=[
            pl.BlockSpec(
                (LOG, N_ATTRIBUTES, PACK_BV), lambda j: (0, 0, block_lo + j)
            )
        ],
        out_specs=pl.BlockSpec((N_ATTRIBUTES, PACK_BV), lambda j: (0, j)),
        out_shape=jax.ShapeDtypeStruct((N_ATTRIBUTES, n_values), jnp.int32),
        compiler_params=pltpu.CompilerParams(
            dimension_semantics=("arbitrary",)
        ),
    )(tab3)


CHUNK = 8192


def _sc_lookup_part(packed, x_t, base, n_values):
    mesh = plsc.VectorSubcoreMesh(core_axis_name="c", subcore_axis_name="s")

    @functools.partial(
        pl.kernel,
        mesh=mesh,
        out_type=jax.ShapeDtypeStruct((N_ATTRIBUTES, BATCH), jnp.int32),
        compiler_params=pltpu.CompilerParams(
            use_tc_tiling_on_sc=False, needs_layout_passes=False
        ),
        scratch_types=[
            pltpu.VMEM((n_values,), jnp.int32),
            pltpu.VMEM((CHUNK,), jnp.int32),
            pltpu.VMEM((CHUNK,), jnp.int32),
            pltpu.VMEM((CHUNK,), jnp.int32),
            pltpu.SemaphoreType.DMA,
            pltpu.SemaphoreType.DMA,
            pltpu.SemaphoreType.DMA,
            pltpu.SemaphoreType.DMA,
        ],
    )
    def k(tab_hbm, idx_hbm, out_hbm, tab_v, idx_v0, idx_v1, out_v,
          sem_t, sem_i0, sem_i1, sem_o):
        wid = lax.axis_index("s") * NUM_CORES + lax.axis_index("c")

        def gather_chunk(src_v, dst_v):
            @pl.loop(0, CHUNK, step=128)
            def _(i):
                for u in range(8):
                    o = i + 16 * u
                    idx = src_v[pl.ds(o, 16)] - base
                    m = (idx >= 0) & (idx < n_values)
                    cl = jnp.where(m, idx, 0)
                    g = plsc.load_gather(tab_v, [cl])
                    dst_v[pl.ds(o, 16)] = jnp.where(m, g, 0)

        @pl.when(wid < N_ATTRIBUTES)
        def _():
            t_cp = pltpu.async_copy(tab_hbm.at[wid], tab_v, sem_t)
            i_cp0 = pltpu.async_copy(
                idx_hbm.at[wid, pl.ds(0, CHUNK)], idx_v0, sem_i0)
            i_cp1 = pltpu.async_copy(
                idx_hbm.at[wid, pl.ds(CHUNK, CHUNK)], idx_v1, sem_i1)
            t_cp.wait()
            i_cp0.wait()
            gather_chunk(idx_v0, out_v)
            o_cp0 = pltpu.async_copy(
                out_v, out_hbm.at[wid, pl.ds(0, CHUNK)], sem_o)
            i_cp1.wait()
            gather_chunk(idx_v1, idx_v0)
            o_cp0.wait()
            o_cp1 = pltpu.async_copy(
                idx_v0, out_hbm.at[wid, pl.ds(CHUNK, CHUNK)], sem_o)
            o_cp1.wait()

    return k(packed, x_t)


UNPACK_BV = 2048
UNPACK_NBLK = BATCH // UNPACK_BV


def _unpack(pc_lo, pc_hi):
    def body(lo_ref, hi_ref, code_ref):
        shift = lax.broadcasted_iota(jnp.int32, (LOG, UNPACK_BV), 0)
        for i in range(N_ATTRIBUTES):
            p = lo_ref[i] + hi_ref[i]
            bits = (jnp.broadcast_to(p[None, :], (LOG, UNPACK_BV)) >> shift) & 1
            code_ref[pl.ds(i * LOG, LOG), :] = bits + 1

    in_spec = pl.BlockSpec((N_ATTRIBUTES, UNPACK_BV), lambda j: (0, j))
    return pl.pallas_call(
        body,
        grid=(UNPACK_NBLK,),
        in_specs=[in_spec, in_spec],
        out_specs=pl.BlockSpec((D_OUT, UNPACK_BV), lambda j: (0, j)),
        out_shape=jax.ShapeDtypeStruct((D_OUT, BATCH), jnp.int32),
        compiler_params=pltpu.CompilerParams(
            dimension_semantics=("arbitrary",)
        ),
    )(pc_lo, pc_hi)


def _zeros2():
    def body(z1_ref, z2_ref):
        z1_ref[...] = jnp.zeros_like(z1_ref)
        z2_ref[...] = jnp.zeros_like(z2_ref)

    spec = pl.BlockSpec((D_OUT, UNPACK_BV), lambda j: (0, j))
    return pl.pallas_call(
        body,
        grid=(UNPACK_NBLK,),
        out_specs=[spec, spec],
        out_shape=[
            jax.ShapeDtypeStruct((D_OUT, BATCH), jnp.float32),
            jax.ShapeDtypeStruct((D_OUT, BATCH), jnp.float32),
        ],
        compiler_params=pltpu.CompilerParams(
            dimension_semantics=("arbitrary",)
        ),
    )()


def kernel(x, tables):
    tab3 = jnp.transpose(tables, (2, 0, 1))
    x_t = jnp.transpose(x, (1, 0))
    packed_lo = _pack(tab3, 0, V_LO)
    pc_lo = _sc_lookup_part(packed_lo, x_t, 0, V_LO)
    packed_hi = _pack(tab3, SPLIT_BLOCKS, V_HI)
    pc_hi = _sc_lookup_part(packed_hi, x_t, V_SPLIT, V_HI)
    codes_fm = _unpack(pc_lo, pc_hi)
    z1, z2 = _zeros2()
    return (codes_fm.T, z1.T, z2.T)
